# baseline (speedup 1.0000x reference)
import functools

import jax
import jax.numpy as jnp
from jax import lax
from jax.experimental import pallas as pl
from jax.experimental.pallas import tpu as pltpu
from jax.experimental.pallas import tpu_sc as plsc

EMBED = 128
GB = 128
NBUF = 6

_info = plsc.get_sparse_core_info()
NC, NS = _info.num_cores, _info.num_subcores
NW = NC * NS


def _span(n):
    s = (-(-n // NW) + 7) // 8 * 8
    assert (n - s) % 8 == 0 and s % 8 == 0
    return s


def _build_sc(n):
    span = _span(n)
    mesh = plsc.VectorSubcoreMesh(core_axis_name="c", subcore_axis_name="s")

    @functools.partial(
        pl.kernel,
        mesh=mesh,
        out_type=jax.ShapeDtypeStruct((n, EMBED), jnp.float32),
        scratch_types=[
            pltpu.VMEM((span,), jnp.int32),
            pltpu.VMEM((NBUF, GB, EMBED), jnp.float32),
            pltpu.VMEM_SHARED((3, EMBED), jnp.float32),
        ]
        + [pltpu.SemaphoreType.DMA] * NBUF
        + [pltpu.SemaphoreType.DMA] * NBUF,
    )
    def k(x, table, o, idx_v, rows_v, table_s, *sems):
        gsems, ssems = sems[:NBUF], sems[NBUF:]
        wid = lax.axis_index("s") * NC + lax.axis_index("c")

        @pl.when(lax.axis_index("s") == 0)
        def _():
            pltpu.sync_copy(table, table_s)

        base = jnp.minimum(wid * span, n - span)
        pltpu.sync_copy(x.at[pl.ds(base, span)], idx_v)

        plsc.subcore_barrier()

        n_ch = -(-span // GB)
        offs = [min(c * GB, span - GB) for c in range(n_ch)]

        def fire_gather(ci):
            return pltpu.async_copy(
                table_s.at[idx_v.at[pl.ds(offs[ci], GB)]],
                rows_v.at[ci % NBUF],
                gsems[ci % NBUF],
            )

        gh = [None] * NBUF
        sh = [None] * NBUF
        for ci in range(min(NBUF, n_ch)):
            gh[ci] = fire_gather(ci)
        for ci in range(n_ch):
            b = ci % NBUF
            gh[b].wait()
            sh[b] = pltpu.async_copy(
                rows_v.at[b], o.at[pl.ds(base + offs[ci], GB)], ssems[b]
            )
            if ci + NBUF < n_ch:
                sh[b].wait()
                gh[b] = fire_gather(ci + NBUF)
        for ci in range(max(0, n_ch - NBUF), n_ch):
            sh[ci % NBUF].wait()

    return k


def _build_tc(n, blk):
    assert n % blk == 0

    def body(idx_ref, tab_ref, o_ref):
        idxb = idx_ref[...]
        t0 = tab_ref[0:1, :]
        t1 = tab_ref[1:2, :]
        t2 = tab_ref[2:3, :]
        o_ref[...] = jnp.where(idxb == 0, t0, jnp.where(idxb == 1, t1, t2))

    return pl.pallas_call(
        body,
        grid=(n // blk,),
        in_specs=[
            pl.BlockSpec((blk, 1), lambda i: (i, 0)),
            pl.BlockSpec((3, EMBED), lambda i: (0, 0)),
        ],
        out_specs=pl.BlockSpec((blk, EMBED), lambda i: (i, 0)),
        out_shape=jax.ShapeDtypeStruct((n, EMBED), jnp.float32),
    )


_sc_user = _build_sc(100000)
_tc_50k = _build_tc(50000, 2000)


def kernel(x_user, x_item, x_category, table):
    ou = _sc_user(x_user.astype(jnp.int32), table)
    oi = _tc_50k(x_item.astype(jnp.int32).reshape(-1, 1), table)
    oc = _tc_50k(x_category.astype(jnp.int32).reshape(-1, 1), table)
    return (ou, ou, oi, oi, oc, oc)


# speedup vs baseline: 1.0105x; 1.0105x over previous
import functools

import jax
import jax.numpy as jnp
from jax import lax
from jax.experimental import pallas as pl
from jax.experimental.pallas import tpu as pltpu
from jax.experimental.pallas import tpu_sc as plsc

EMBED = 128
GB = 128
NBUF = 6

_info = plsc.get_sparse_core_info()
NC, NS = _info.num_cores, _info.num_subcores
NW = NC * NS


def _span(n):
    s = (-(-n // NW) + 7) // 8 * 8
    assert (n - s) % 8 == 0 and s % 8 == 0
    return s


def _build_sc(n):
    span = _span(n)
    mesh = plsc.VectorSubcoreMesh(core_axis_name="c", subcore_axis_name="s")

    @functools.partial(
        pl.kernel,
        mesh=mesh,
        out_type=jax.ShapeDtypeStruct((n, EMBED), jnp.float32),
        scratch_types=[
            pltpu.VMEM((span,), jnp.int32),
            pltpu.VMEM((NBUF, GB, EMBED), jnp.float32),
            pltpu.VMEM_SHARED((3, EMBED), jnp.float32),
        ]
        + [pltpu.SemaphoreType.DMA] * NBUF
        + [pltpu.SemaphoreType.DMA] * NBUF,
    )
    def k(x, table, o, idx_v, rows_v, table_s, *sems):
        gsems, ssems = sems[:NBUF], sems[NBUF:]
        wid = lax.axis_index("s") * NC + lax.axis_index("c")

        @pl.when(lax.axis_index("s") == 0)
        def _():
            pltpu.sync_copy(table, table_s)

        base = jnp.minimum(wid * span, n - span)
        pltpu.sync_copy(x.at[pl.ds(base, span)], idx_v)

        plsc.subcore_barrier()

        n_ch = -(-span // GB)
        offs = [min(c * GB, span - GB) for c in range(n_ch)]

        def fire_gather(ci):
            return pltpu.async_copy(
                table_s.at[idx_v.at[pl.ds(offs[ci], GB)]],
                rows_v.at[ci % NBUF],
                gsems[ci % NBUF],
            )

        gh = [None] * NBUF
        sh = [None] * NBUF
        for ci in range(min(NBUF, n_ch)):
            gh[ci] = fire_gather(ci)
        for ci in range(n_ch):
            b = ci % NBUF
            gh[b].wait()
            sh[b] = pltpu.async_copy(
                rows_v.at[b], o.at[pl.ds(base + offs[ci], GB)], ssems[b]
            )
            if ci + NBUF < n_ch:
                sh[b].wait()
                gh[b] = fire_gather(ci + NBUF)
        for ci in range(max(0, n_ch - NBUF), n_ch):
            sh[ci % NBUF].wait()

    return k


def _build_tc(n, blk):
    assert n % blk == 0

    def body(idx_ref, tab_ref, o_ref):
        idxb = idx_ref[...]
        oh = (idxb == lax.broadcasted_iota(jnp.int32, (1, 8), 1)).astype(
            jnp.float32
        )
        o_ref[...] = jnp.dot(
            oh, tab_ref[...], preferred_element_type=jnp.float32
        )

    return pl.pallas_call(
        body,
        grid=(n // blk,),
        in_specs=[
            pl.BlockSpec((blk, 1), lambda i: (i, 0)),
            pl.BlockSpec((8, EMBED), lambda i: (0, 0)),
        ],
        out_specs=pl.BlockSpec((blk, EMBED), lambda i: (i, 0)),
        out_shape=jax.ShapeDtypeStruct((n, EMBED), jnp.float32),
    )


_sc_user = _build_sc(100000)
_tc_50k = _build_tc(50000, 2000)


def kernel(x_user, x_item, x_category, table):
    table8 = jnp.zeros((8, EMBED), jnp.float32).at[:3].set(table)
    ou = _sc_user(x_user.astype(jnp.int32), table)
    oi = _tc_50k(x_item.astype(jnp.int32).reshape(-1, 1), table8)
    oc = _tc_50k(x_category.astype(jnp.int32).reshape(-1, 1), table8)
    return (ou, ou, oi, oi, oc, oc)


# speedup vs baseline: 1.3740x; 1.3598x over previous
import functools

import jax
import jax.numpy as jnp
from jax import lax
from jax.experimental import pallas as pl
from jax.experimental.pallas import tpu as pltpu
from jax.experimental.pallas import tpu_sc as plsc

EMBED = 128
GB = 128
NBUF = 6

_info = plsc.get_sparse_core_info()
NC, NS = _info.num_cores, _info.num_subcores
NW = NC * NS


def _span(n):
    s = (-(-n // NW) + 7) // 8 * 8
    assert (n - s) % 8 == 0 and s % 8 == 0
    return s


def _build_sc(n):
    span = _span(n)
    mesh = plsc.VectorSubcoreMesh(core_axis_name="c", subcore_axis_name="s")

    @functools.partial(
        pl.kernel,
        mesh=mesh,
        out_type=jax.ShapeDtypeStruct((n, EMBED), jnp.float32),
        scratch_types=[
            pltpu.VMEM((span,), jnp.int32),
            pltpu.VMEM((NBUF, GB, EMBED), jnp.float32),
            pltpu.VMEM_SHARED((3, EMBED), jnp.float32),
        ]
        + [pltpu.SemaphoreType.DMA] * NBUF
        + [pltpu.SemaphoreType.DMA] * NBUF,
    )
    def k(x, table, o, idx_v, rows_v, table_s, *sems):
        gsems, ssems = sems[:NBUF], sems[NBUF:]
        wid = lax.axis_index("s") * NC + lax.axis_index("c")

        @pl.when(lax.axis_index("s") == 0)
        def _():
            pltpu.sync_copy(table, table_s)

        base = jnp.minimum(wid * span, n - span)
        pltpu.sync_copy(x.at[pl.ds(base, span)], idx_v)

        plsc.subcore_barrier()

        n_ch = -(-span // GB)
        offs = [min(c * GB, span - GB) for c in range(n_ch)]

        def fire_gather(ci):
            return pltpu.async_copy(
                table_s.at[idx_v.at[pl.ds(offs[ci], GB)]],
                rows_v.at[ci % NBUF],
                gsems[ci % NBUF],
            )

        gh = [None] * NBUF
        sh = [None] * NBUF
        for ci in range(min(NBUF, n_ch)):
            gh[ci] = fire_gather(ci)
        for ci in range(n_ch):
            b = ci % NBUF
            gh[b].wait()
            sh[b] = pltpu.async_copy(
                rows_v.at[b], o.at[pl.ds(base + offs[ci], GB)], ssems[b]
            )
            if ci + NBUF < n_ch:
                sh[b].wait()
                gh[b] = fire_gather(ci + NBUF)
        for ci in range(max(0, n_ch - NBUF), n_ch):
            sh[ci % NBUF].wait()

    return k


TCB = 2048


def _build_tc(n):
    g = -(-n // TCB)

    def body(idx_ref, tab_ref, o_ref):
        idx_t = idx_ref[...].T
        t0 = tab_ref[0:1, :]
        t1 = tab_ref[1:2, :]
        t2 = tab_ref[2:3, :]
        for r in range(16):
            c = idx_t[:, r : r + 1]
            o_ref[r * 128 : (r + 1) * 128, :] = jnp.where(
                c == 0, t0, jnp.where(c == 1, t1, t2)
            )

    call = pl.pallas_call(
        body,
        grid=(g,),
        in_specs=[
            pl.BlockSpec((16, 128), lambda i: (i, 0)),
            pl.BlockSpec((3, EMBED), lambda i: (0, 0)),
        ],
        out_specs=pl.BlockSpec((TCB, EMBED), lambda i: (i, 0)),
        out_shape=jax.ShapeDtypeStruct((n, EMBED), jnp.float32),
    )

    def run(x, table):
        pad = g * TCB - n
        xp = jnp.concatenate([x, jnp.zeros((pad,), jnp.int32)])
        return call(xp.reshape(g * 16, 128), table)

    return run


_sc_user = _build_sc(100000)
_tc_item = _build_tc(50000)
_tc_cat = _build_tc(50000)


def kernel(x_user, x_item, x_category, table):
    ou = _sc_user(x_user.astype(jnp.int32), table)
    oi = _tc_item(x_item.astype(jnp.int32), table)
    oc = _tc_cat(x_category.astype(jnp.int32), table)
    return (ou, ou, oi, oi, oc, oc)


# speedup vs baseline: 1.4324x; 1.0425x over previous
import functools

import jax
import jax.numpy as jnp
from jax import lax
from jax.experimental import pallas as pl
from jax.experimental.pallas import tpu as pltpu
from jax.experimental.pallas import tpu_sc as plsc

EMBED = 128
GB = 128
NBUF = 7

_info = plsc.get_sparse_core_info()
NC, NS = _info.num_cores, _info.num_subcores
NW = NC * NS


def _span(n):
    s = (-(-n // NW) + 7) // 8 * 8
    assert (n - s) % 8 == 0 and s % 8 == 0
    return s


def _build(n_user, n_item, n_cat):
    ns = (n_user, n_item, n_cat)
    spans = tuple(_span(n) for n in ns)
    seg_offs = (0, spans[0], spans[0] + spans[1])
    idx_total = sum(spans)
    mesh = plsc.VectorSubcoreMesh(core_axis_name="c", subcore_axis_name="s")
    out_types = tuple(
        jax.ShapeDtypeStruct((n, EMBED), jnp.float32) for n in ns
    )

    @functools.partial(
        pl.kernel,
        mesh=mesh,
        out_type=out_types,
        scratch_types=[
            pltpu.VMEM((idx_total,), jnp.int32),
            pltpu.VMEM((NBUF, GB, EMBED), jnp.float32),
            pltpu.VMEM_SHARED((3, EMBED), jnp.float32),
        ]
        + [pltpu.SemaphoreType.DMA] * NBUF
        + [pltpu.SemaphoreType.DMA] * NBUF,
    )
    def k(xu, xi, xc, table, ou, oi, oc, idx_v, rows_v, table_s, *sems):
        gsems, ssems = sems[:NBUF], sems[NBUF:]
        wid = lax.axis_index("s") * NC + lax.axis_index("c")

        @pl.when(lax.axis_index("s") == 0)
        def _():
            pltpu.sync_copy(table, table_s)

        bases = []
        for x, n, span, soff in zip((xu, xi, xc), ns, spans, seg_offs):
            base = jnp.minimum(wid * span, n - span)
            bases.append(base)
            pltpu.sync_copy(
                x.at[pl.ds(base, span)], idx_v.at[pl.ds(soff, span)]
            )

        plsc.subcore_barrier()

        chunks = []
        for o, base, span, soff in zip((ou, oi, oc), bases, spans, seg_offs):
            n_ch = -(-span // GB)
            for c in range(n_ch):
                off = min(c * GB, span - GB)
                chunks.append((o, base, soff + off, off))

        nch = len(chunks)

        def fire_gather(ci):
            _, _, ioff, _ = chunks[ci]
            return pltpu.async_copy(
                table_s.at[idx_v.at[pl.ds(ioff, GB)]],
                rows_v.at[ci % NBUF],
                gsems[ci % NBUF],
            )

        gh = [None] * NBUF
        sh = [None] * NBUF
        for ci in range(min(NBUF, nch)):
            gh[ci] = fire_gather(ci)
        for ci in range(nch):
            b = ci % NBUF
            o, base, _, off = chunks[ci]
            gh[b].wait()
            sh[b] = pltpu.async_copy(
                rows_v.at[b], o.at[pl.ds(base + off, GB)], ssems[b]
            )
            if ci + NBUF < nch:
                sh[b].wait()
                gh[b] = fire_gather(ci + NBUF)
        for ci in range(max(0, nch - NBUF), nch):
            sh[ci % NBUF].wait()

    return k


_embed3 = _build(100000, 50000, 50000)


def kernel(x_user, x_item, x_category, table):
    ou, oi, oc = _embed3(
        x_user.astype(jnp.int32),
        x_item.astype(jnp.int32),
        x_category.astype(jnp.int32),
        table,
    )
    return (ou, ou, oi, oi, oc, oc)
